# Initial kernel scaffold; baseline (speedup 1.0000x reference)
import jax
import jax.numpy as jnp
from jax.experimental import pallas as pl


def kernel(x, conv_weight, bn_gamma, bn_beta, bn_mean, bn_var):
    raise NotImplementedError("write your pallas kernel here")


# speedup vs baseline: 9.9743x; 9.9743x over previous
import jax
import jax.numpy as jnp
from jax.experimental import pallas as pl
from jax.experimental.pallas import tpu as pltpu


def _stem_kernel(w_ref, b_ref, x_ref, o_ref, s1_ref, s2_ref):
    rolled = {}
    for c in range(3):
        for p in range(2):
            for q in range(2):
                t = x_ref[0, c, p, q]
                max_dw = 4 if q == 0 else 3
                for dw in range(max_dw):
                    if dw == 0:
                        r = t
                    else:
                        r = jnp.concatenate([t[:, dw:], t[:, :dw]], axis=1)
                    rolled[(c, p, q, dw)] = r

    rows = []
    for c in range(3):
        for kh in range(7):
            p, dh = kh % 2, kh // 2
            for kw in range(7):
                q, dw = kw % 2, kw // 2
                rows.append(rolled[(c, p, q, dw)][dh:dh + 112, :])
    m = jnp.stack(rows, axis=0).reshape(147, 112 * 128)

    acc = jax.lax.dot_general(
        w_ref[...], m, (((1,), (0,)), ((), ())),
        preferred_element_type=jnp.float32)
    acc = jnp.maximum(acc + b_ref[:, 0:1], 0.0)
    a3 = acc.reshape(64, 112, 128)

    z_row = jnp.zeros((64, 1, 128), jnp.float32)
    up = jnp.concatenate([a3[:, 1:, :], z_row], axis=1)
    dn = jnp.concatenate([z_row, a3[:, :-1, :]], axis=1)
    s1_ref[...] = jnp.maximum(jnp.maximum(a3, up), dn)
    m1 = s1_ref[:, pl.ds(0, 56, 2), :]

    z_col = jnp.zeros((64, 56, 1), jnp.float32)
    lf = jnp.concatenate([m1[:, :, 1:], z_col], axis=2)
    rt = jnp.concatenate([z_col, m1[:, :, :-1]], axis=2)
    m2 = jnp.maximum(jnp.maximum(m1, lf), rt)
    s2_ref[...] = jnp.transpose(m2, (0, 2, 1))
    o_ref[0] = jnp.transpose(s2_ref[:, pl.ds(0, 56, 2), :], (0, 2, 1))


def kernel(x, conv_weight, bn_gamma, bn_beta, bn_mean, bn_var):
    eps = 1e-5
    B, C_in, H, W = x.shape
    C_out = conv_weight.shape[0]

    scale = bn_gamma / jnp.sqrt(bn_var + eps)
    w_f = (conv_weight.astype(jnp.float32)
           * scale[:, None, None, None]).reshape(C_out, C_in * 49)
    bias = (bn_beta - bn_mean * scale).astype(jnp.float32)
    b_f = jnp.broadcast_to(bias[:, None], (C_out, 128))

    xp = jnp.pad(x.astype(jnp.float32), ((0, 0), (0, 0), (3, 3), (3, 3)))
    phases = jnp.stack(
        [jnp.stack([xp[:, :, p::2, q::2] for q in range(2)], axis=2)
         for p in range(2)], axis=2)
    phases = jnp.pad(phases, ((0, 0), (0, 0), (0, 0), (0, 0), (0, 5), (0, 13)))

    out = pl.pallas_call(
        _stem_kernel,
        out_shape=jax.ShapeDtypeStruct((B, C_out, 56, 56), jnp.float32),
        grid_spec=pltpu.PrefetchScalarGridSpec(
            num_scalar_prefetch=0,
            grid=(B,),
            in_specs=[
                pl.BlockSpec((C_out, C_in * 49), lambda b: (0, 0)),
                pl.BlockSpec((C_out, 128), lambda b: (0, 0)),
                pl.BlockSpec((1, 3, 2, 2, 120, 128), lambda b: (b, 0, 0, 0, 0, 0)),
            ],
            out_specs=pl.BlockSpec((1, C_out, 56, 56), lambda b: (b, 0, 0, 0)),
            scratch_shapes=[
                pltpu.VMEM((C_out, 112, 128), jnp.float32),
                pltpu.VMEM((C_out, 128, 56), jnp.float32),
            ],
        ),
        compiler_params=pltpu.CompilerParams(dimension_semantics=("parallel",)),
    )(w_f, b_f, phases)
    return out
